# baseline (speedup 1.0000x reference)
import jax
import jax.numpy as jnp
from jax.experimental import pallas as pl

N_NODES = 100000
N_SPECIES = 64
ROWS = N_NODES // 2
BLOCK = 1000


def _onehot_body(idx_ref, out_ref):
    i2 = idx_ref[...]
    lo = i2[:, 0:1]
    hi = i2[:, 1:2]
    lane = jax.lax.broadcasted_iota(jnp.int32, (BLOCK, 2 * N_SPECIES), 1)
    t = jnp.where(lane < N_SPECIES, lo, hi)
    out_ref[...] = (t == (lane & (N_SPECIES - 1))).astype(jnp.float32)


def kernel(atom_types):
    idx2 = atom_types.reshape(ROWS, 2)
    out128 = pl.pallas_call(
        _onehot_body,
        grid=(ROWS // BLOCK,),
        in_specs=[pl.BlockSpec((BLOCK, 2), lambda i: (i, 0))],
        out_specs=pl.BlockSpec((BLOCK, 2 * N_SPECIES), lambda i: (i, 0)),
        out_shape=jax.ShapeDtypeStruct((ROWS, 2 * N_SPECIES), jnp.float32),
    )(idx2)
    return out128.reshape(N_NODES, N_SPECIES)


# speedup vs baseline: 1.2094x; 1.2094x over previous
import jax
import jax.numpy as jnp
from jax.experimental import pallas as pl

N_NODES = 100000
N_SPECIES = 64
BLOCK = 1000


def _onehot_body(idx_ref, out_ref):
    t = idx_ref[...]
    cols = jax.lax.broadcasted_iota(jnp.int32, (BLOCK, N_SPECIES), 1)
    out_ref[...] = (t == cols).astype(jnp.float32)


def kernel(atom_types):
    grid = (N_NODES // BLOCK,)
    return pl.pallas_call(
        _onehot_body,
        grid=grid,
        in_specs=[pl.BlockSpec((BLOCK, 1), lambda i: (i, 0))],
        out_specs=pl.BlockSpec((BLOCK, N_SPECIES), lambda i: (i, 0)),
        out_shape=jax.ShapeDtypeStruct((N_NODES, N_SPECIES), jnp.float32),
    )(atom_types)


# speedup vs baseline: 16.3488x; 13.5183x over previous
import jax
import jax.numpy as jnp
from jax.experimental import pallas as pl

N_NODES = 100000
N_SPECIES = 64
SUB = 8


def _onehot_t_body(idx_ref, out_ref):
    i = pl.program_id(0)
    row = idx_ref[...]
    sp = jax.lax.broadcasted_iota(jnp.int32, (SUB, N_NODES), 0) + SUB * i
    out_ref[...] = (row == sp).astype(jnp.float32)


def kernel(atom_types):
    idx_t = atom_types.T
    out_t = pl.pallas_call(
        _onehot_t_body,
        grid=(N_SPECIES // SUB,),
        in_specs=[pl.BlockSpec((1, N_NODES), lambda i: (0, 0))],
        out_specs=pl.BlockSpec((SUB, N_NODES), lambda i: (i, 0)),
        out_shape=jax.ShapeDtypeStruct((N_SPECIES, N_NODES), jnp.float32),
    )(idx_t)
    return out_t.T
